# Initial kernel scaffold; baseline (speedup 1.0000x reference)
import jax
import jax.numpy as jnp
from jax.experimental import pallas as pl


def kernel(label_indices, tokens):
    raise NotImplementedError("write your pallas kernel here")


# speedup vs baseline: 1.6853x; 1.6853x over previous
import functools

import jax
import jax.numpy as jnp
from jax import lax
from jax.experimental import pallas as pl
from jax.experimental.pallas import tpu as pltpu
from jax.experimental.pallas import tpu_sc as plsc

_L = 16


@functools.cache
def _build(B, C, POOL, TOK, CH):
    info = plsc.get_sparse_core_info()
    NC, NS = info.num_cores, info.num_subcores
    NW = NC * NS
    nsel = C - 1
    row = TOK * CH
    b_per_w = B // NW
    rows_per_w = b_per_w * nsel
    K = 32
    nchunks = (rows_per_w + K - 1) // K
    assert rows_per_w % K == 0 and rows_per_w % _L == 0

    mesh = plsc.VectorSubcoreMesh(core_axis_name="c", subcore_axis_name="s")

    @functools.partial(
        pl.kernel,
        out_type=jax.ShapeDtypeStruct((B * nsel, row), jnp.float32),
        mesh=mesh,
        compiler_params=pltpu.CompilerParams(needs_layout_passes=False),
        scratch_types=[
            pltpu.VMEM((b_per_w, 2 * _L), jnp.int32),
            pltpu.VMEM((rows_per_w,), jnp.int32),
            pltpu.VMEM((K, row), jnp.float32),
            pltpu.VMEM((K, row), jnp.float32),
            pltpu.SemaphoreType.DMA,
            pltpu.SemaphoreType.DMA,
        ],
    )
    def kfn(lp_hbm, table_hbm, out_hbm, lp_v, idx_v, buf0, buf1, sem0, sem1):
        wid = lax.axis_index("s") * NC + lax.axis_index("c")
        sbase = wid * b_per_w

        pltpu.sync_copy(lp_hbm.at[pl.ds(sbase, b_per_w)], lp_v)

        iota = lax.iota(jnp.int32, _L)
        zeros = iota * 0
        ones = zeros + 1
        for i in range(rows_per_w // _L):
            idx_v[pl.ds(i * _L, _L)] = ones

        for smp in range(b_per_w):
            ch0 = lp_v[smp, pl.ds(0, _L)]
            ch1 = lp_v[smp, pl.ds(_L, _L)]
            m0 = ch0 != zeros
            m1 = ch1 != zeros
            m0i = jnp.where(m0, ones, zeros)
            m1i = jnp.where(m1, ones, zeros)
            c0 = plsc.cumsum(m0i)
            n0 = jnp.sum(m0i)
            n0v = lax.broadcast_in_dim(n0, (_L,), ())
            c1 = plsc.cumsum(m1i)
            base = smp * nsel - 1
            plsc.store_scatter(idx_v, [c0 + base], iota + 1, mask=m0)
            plsc.store_scatter(idx_v, [c1 + n0v + base], iota + (_L + 1),
                               mask=m1)

        row0 = wid * rows_per_w
        bufs = (buf0, buf1)
        sems = (sem0, sem1)

        def gather(k):
            return pltpu.async_copy(
                table_hbm.at[idx_v.at[pl.ds(k * K, K)]], bufs[k % 2],
                sems[k % 2])

        pending = gather(0)
        for k in range(nchunks):
            nxt = gather(k + 1) if k + 1 < nchunks else None
            pending.wait()
            pltpu.sync_copy(bufs[k % 2], out_hbm.at[pl.ds(row0 + k * K, K)])
            pending = nxt

    return kfn


def kernel(label_indices, tokens):
    B, C = label_indices.shape
    POOL, TOK, CH = tokens.shape
    nsel = C - 1
    lp = jnp.pad(label_indices[:, 1:].astype(jnp.int32),
                 ((0, 0), (0, 2 * _L - nsel)))
    table = tokens.reshape(POOL, TOK * CH)
    out = _build(B, C, POOL, TOK, CH)(lp, table)
    return out.reshape(B, nsel * TOK, CH)
